# baseline (speedup 1.0000x reference)
import functools

import jax
import jax.numpy as jnp
from jax import lax
from jax.experimental import pallas as pl
from jax.experimental.pallas import tpu as pltpu
from jax.experimental.pallas import tpu_sc as plsc

BATCH = 16384
EMBED_DIM = 32
VOCAB1 = 1000001
GROUPS = (VOCAB1 + 3) // 4
GROUP_W = 4 * EMBED_DIM

_info = plsc.get_sparse_core_info()
_NC, _NS, _NL = _info.num_cores, _info.num_subcores, _info.num_lanes
_NW = _NC * _NS
_B_PER_W = BATCH // _NW
_CHUNK = 128
_N_CHUNK = _B_PER_W // _CHUNK


def _make_gather():
    mesh = plsc.VectorSubcoreMesh(core_axis_name="c", subcore_axis_name="s")

    @functools.partial(
        pl.kernel,
        mesh=mesh,
        out_type=jax.ShapeDtypeStruct((BATCH, EMBED_DIM), jnp.float32),
        scratch_types=[
            pltpu.VMEM((_B_PER_W,), jnp.int32),
            pltpu.VMEM((_B_PER_W,), jnp.int32),
            pltpu.VMEM((2, _CHUNK, GROUP_W), jnp.float32),
            pltpu.VMEM((_B_PER_W, EMBED_DIM), jnp.float32),
            pltpu.SemaphoreType.DMA,
            pltpu.SemaphoreType.DMA,
        ],
        compiler_params=pltpu.CompilerParams(needs_layout_passes=False),
    )
    def gather_kernel(idx_hbm, view_hbm, out_hbm, idx_v, gidx_v, grp_v,
                      rows_v, sem0, sem1):
        wid = lax.axis_index("s") * _NC + lax.axis_index("c")
        base = wid * _B_PER_W
        pltpu.sync_copy(idx_hbm.at[pl.ds(base, _B_PER_W)], idx_v)
        for k in range(_B_PER_W // _NL):
            ids = idx_v[pl.ds(k * _NL, _NL)]
            gidx_v[pl.ds(k * _NL, _NL)] = ids >> 2
        sems = (sem0, sem1)

        def fire(c):
            return pltpu.async_copy(
                view_hbm.at[gidx_v.at[pl.ds(c * _CHUNK, _CHUNK)]],
                grp_v.at[c % 2],
                sems[c % 2],
            )

        lanes = lax.iota(jnp.int32, _NL)
        copies = [fire(0), fire(1)]
        for c in range(_N_CHUNK):
            copies[c % 2].wait()
            buf = grp_v.at[c % 2]
            for k in range(_CHUNK // _NL):
                ids = idx_v[pl.ds(c * _CHUNK + k * _NL, _NL)]
                sub = ids & jnp.int32(3)
                lrows = k * _NL + lanes
                rows = c * _CHUNK + k * _NL + lanes
                col0 = sub * jnp.int32(EMBED_DIM)
                for j in range(EMBED_DIM):
                    jv = jnp.full((_NL,), j, jnp.int32)
                    col = plsc.load_gather(buf, [lrows, col0 + j])
                    plsc.store_scatter(rows_v, [rows, jv], col)
            if c + 2 < _N_CHUNK:
                copies[c % 2] = fire(c + 2)
        pltpu.sync_copy(rows_v, out_hbm.at[pl.ds(base, _B_PER_W)])

    return gather_kernel


_gather = _make_gather()


def kernel(user_id, table):
    view = jnp.pad(table, ((0, 3), (0, 0))).reshape(GROUPS, GROUP_W)
    return _gather(user_id, view)
